# baseline (speedup 1.0000x reference)
import functools

import jax
import jax.numpy as jnp
from jax import lax
from jax.experimental import pallas as pl
from jax.experimental.pallas import tpu as pltpu
from jax.experimental.pallas import tpu_sc as plsc

_B, _S = 4, 4096
_L = 16
_NW = 16
_WPR = _NW // _B
_C = _S // _WPR
_NI = _C // _L

_GATHER_DNUMS = lax.GatherDimensionNumbers(
    offset_dims=(), collapsed_slice_dims=(0,), start_index_map=(0,)
)


def _shuffle(x, idx):
    return lax.gather(
        x, idx[:, None], dimension_numbers=_GATHER_DNUMS, slice_sizes=(1,),
        mode=lax.GatherScatterMode.PROMISE_IN_BOUNDS,
    )


def _combine(v1, i1, v2, i2):
    take = (v2 > v1) | ((v2 == v1) & (i2 < i1))
    return jnp.where(take, v2, v1), jnp.where(take, i2, i1)


_mesh = plsc.VectorSubcoreMesh(
    core_axis_name="c", subcore_axis_name="s", num_cores=1, num_subcores=_NW
)


@functools.partial(
    pl.kernel,
    out_type=(
        jax.ShapeDtypeStruct((_B, _S), jnp.float32),
        jax.ShapeDtypeStruct((_B, _S), jnp.int32),
    ),
    mesh=_mesh,
    scratch_types=[
        pltpu.VMEM((_C,), jnp.float32),
        pltpu.VMEM((_C,), jnp.float32),
        pltpu.VMEM((_C,), jnp.float32),
        pltpu.VMEM((_C,), jnp.int32),
        pltpu.VMEM((_L,), jnp.float32),
        pltpu.VMEM((_L,), jnp.int32),
        pltpu.VMEM((_WPR * _L,), jnp.float32),
        pltpu.VMEM((_WPR * _L,), jnp.int32),
        pltpu.VMEM((_L,), jnp.int32),
        pltpu.VMEM_SHARED((_NW * _L,), jnp.float32),
        pltpu.VMEM_SHARED((_NW * _L,), jnp.int32),
        pltpu.SemaphoreType.DMA,
        pltpu.SemaphoreType.DMA,
    ],
)
def _sc_kernel(logits_hbm, mask_hbm, masked_hbm, markers_hbm,
               lg_v, mk_v, out_v, mark_v, cmax_v, cidx_v,
               rmax_v, ridx_v, ixs_v, sh_max, sh_idx, sem_a, sem_b):
    wid = lax.axis_index("s")
    row = wid // _WPR
    col0 = (wid % _WPR) * _C

    ca = pltpu.async_copy(logits_hbm.at[row, pl.ds(col0, _C)], lg_v, sem_a)
    cb = pltpu.async_copy(mask_hbm.at[row, pl.ds(col0, _C)], mk_v, sem_b)
    ca.wait()
    cb.wait()

    lanes = lax.iota(jnp.int32, _L)
    zero_i = jnp.zeros((_L,), jnp.int32)
    one_i = jnp.ones((_L,), jnp.int32)

    def body(i, carry):
        vmax, ibest = carry
        lv = lg_v[pl.ds(i * _L, _L)]
        mv = mk_v[pl.ds(i * _L, _L)]
        masked = lv * (1.0 - mv) - 1e30 * mv
        out_v[pl.ds(i * _L, _L)] = masked
        upd = masked > vmax
        vmax = jnp.where(upd, masked, vmax)
        ibest = jnp.where(upd, jnp.broadcast_to(i, (_L,)), ibest)
        return vmax, ibest

    vmax, ibest = lax.fori_loop(
        0, _NI, body,
        (jnp.full((_L,), -jnp.inf, jnp.float32), zero_i),
        unroll=8,
    )

    cmax_v[...] = vmax
    cidx_v[...] = jnp.broadcast_to(col0, (_L,)) + ibest * _L + lanes
    pltpu.sync_copy(cmax_v, sh_max.at[pl.ds(wid * _L, _L)])
    pltpu.sync_copy(cidx_v, sh_idx.at[pl.ds(wid * _L, _L)])
    plsc.subcore_barrier()

    pltpu.sync_copy(sh_max.at[pl.ds(row * _WPR * _L, _WPR * _L)], rmax_v)
    pltpu.sync_copy(sh_idx.at[pl.ds(row * _WPR * _L, _WPR * _L)], ridx_v)
    v, ix = rmax_v[pl.ds(0, _L)], ridx_v[pl.ds(0, _L)]
    for k in range(1, _WPR):
        v, ix = _combine(v, ix, rmax_v[pl.ds(k * _L, _L)], ridx_v[pl.ds(k * _L, _L)])

    for shift in (8, 4, 2, 1):
        partner = jnp.bitwise_xor(lanes, shift)
        v, ix = _combine(v, ix, _shuffle(v, partner), _shuffle(ix, partner))

    ixs_v[...] = ix

    def mark_body(i, _):
        flat_i = jnp.broadcast_to(col0 + i * _L, (_L,)) + lanes
        mark_v[pl.ds(i * _L, _L)] = jnp.where(flat_i == ixs_v[...], one_i, zero_i)
        return 0

    lax.fori_loop(0, _NI, mark_body, 0, unroll=8)

    pltpu.sync_copy(out_v, masked_hbm.at[row, pl.ds(col0, _C)])
    pltpu.sync_copy(mark_v, markers_hbm.at[row, pl.ds(col0, _C)])


def kernel(input_ids, logits, logits_mask):
    masked, markers = _sc_kernel(logits, logits_mask)
    return masked, markers.astype(input_ids.dtype)


# speedup vs baseline: 1.0150x; 1.0150x over previous
import functools

import jax
import jax.numpy as jnp
from jax import lax
from jax.experimental import pallas as pl
from jax.experimental.pallas import tpu as pltpu
from jax.experimental.pallas import tpu_sc as plsc

_B, _S = 4, 4096
_L = 16
_NW = 16
_WPR = _NW // _B
_C = _S // _WPR
_NI = _C // _L

_GATHER_DNUMS = lax.GatherDimensionNumbers(
    offset_dims=(), collapsed_slice_dims=(0,), start_index_map=(0,)
)


def _shuffle(x, idx):
    return lax.gather(
        x, idx[:, None], dimension_numbers=_GATHER_DNUMS, slice_sizes=(1,),
        mode=lax.GatherScatterMode.PROMISE_IN_BOUNDS,
    )


def _combine(v1, i1, v2, i2):
    take = (v2 > v1) | ((v2 == v1) & (i2 < i1))
    return jnp.where(take, v2, v1), jnp.where(take, i2, i1)


_mesh = plsc.VectorSubcoreMesh(
    core_axis_name="c", subcore_axis_name="s", num_cores=1, num_subcores=_NW
)


@functools.partial(
    pl.kernel,
    out_type=(
        jax.ShapeDtypeStruct((_B, _S), jnp.float32),
        jax.ShapeDtypeStruct((_B, _S), jnp.int32),
    ),
    mesh=_mesh,
    scratch_types=[
        pltpu.VMEM((_C,), jnp.float32),
        pltpu.VMEM((_C,), jnp.float32),
        pltpu.VMEM((_C,), jnp.float32),
        pltpu.VMEM((_C,), jnp.int32),
        pltpu.VMEM((_L,), jnp.float32),
        pltpu.VMEM((_L,), jnp.int32),
        pltpu.VMEM((_WPR * _L,), jnp.float32),
        pltpu.VMEM((_WPR * _L,), jnp.int32),
        pltpu.VMEM((_L,), jnp.int32),
        pltpu.VMEM_SHARED((_NW * _L,), jnp.float32),
        pltpu.VMEM_SHARED((_NW * _L,), jnp.int32),
        pltpu.SemaphoreType.DMA,
        pltpu.SemaphoreType.DMA,
    ],
)
def _sc_kernel(logits_hbm, mask_hbm, masked_hbm, markers_hbm,
               lg_v, mk_v, out_v, mark_v, cmax_v, cidx_v,
               rmax_v, ridx_v, ixs_v, sh_max, sh_idx, sem_a, sem_b):
    wid = lax.axis_index("s")
    row = wid // _WPR
    col0 = (wid % _WPR) * _C

    ca = pltpu.async_copy(logits_hbm.at[row, pl.ds(col0, _C)], lg_v, sem_a)
    cb = pltpu.async_copy(mask_hbm.at[row, pl.ds(col0, _C)], mk_v, sem_b)
    ca.wait()
    cb.wait()

    lanes = lax.iota(jnp.int32, _L)
    zero_i = jnp.zeros((_L,), jnp.int32)
    one_i = jnp.ones((_L,), jnp.int32)

    def body(i, carry):
        vmax, ibest = carry
        lv = lg_v[pl.ds(i * _L, _L)]
        mv = mk_v[pl.ds(i * _L, _L)]
        masked = lv * (1.0 - mv) - 1e30 * mv
        out_v[pl.ds(i * _L, _L)] = masked
        upd = masked > vmax
        vmax = jnp.where(upd, masked, vmax)
        ibest = jnp.where(upd, jnp.broadcast_to(i, (_L,)), ibest)
        return vmax, ibest

    vmax, ibest = lax.fori_loop(
        0, _NI, body,
        (jnp.full((_L,), -jnp.inf, jnp.float32), zero_i),
        unroll=8,
    )

    pltpu.async_copy(out_v, masked_hbm.at[row, pl.ds(col0, _C)], sem_a)

    cmax_v[...] = vmax
    cidx_v[...] = jnp.broadcast_to(col0, (_L,)) + ibest * _L + lanes
    pltpu.sync_copy(cmax_v, sh_max.at[pl.ds(wid * _L, _L)])
    pltpu.sync_copy(cidx_v, sh_idx.at[pl.ds(wid * _L, _L)])
    plsc.subcore_barrier()

    pltpu.sync_copy(sh_max.at[pl.ds(row * _WPR * _L, _WPR * _L)], rmax_v)
    pltpu.sync_copy(sh_idx.at[pl.ds(row * _WPR * _L, _WPR * _L)], ridx_v)
    v, ix = rmax_v[pl.ds(0, _L)], ridx_v[pl.ds(0, _L)]
    for k in range(1, _WPR):
        v, ix = _combine(v, ix, rmax_v[pl.ds(k * _L, _L)], ridx_v[pl.ds(k * _L, _L)])

    for shift in (8, 4, 2, 1):
        partner = jnp.bitwise_xor(lanes, shift)
        v, ix = _combine(v, ix, _shuffle(v, partner), _shuffle(ix, partner))

    ixs_v[...] = ix

    def mark_body(i, _):
        flat_i = jnp.broadcast_to(col0 + i * _L, (_L,)) + lanes
        mark_v[pl.ds(i * _L, _L)] = jnp.where(flat_i == ixs_v[...], one_i, zero_i)
        return 0

    lax.fori_loop(0, _NI, mark_body, 0, unroll=8)

    pltpu.async_copy(mark_v, markers_hbm.at[row, pl.ds(col0, _C)], sem_b)
    pltpu.make_async_copy(out_v, masked_hbm.at[row, pl.ds(col0, _C)], sem_a).wait()
    pltpu.make_async_copy(mark_v, markers_hbm.at[row, pl.ds(col0, _C)], sem_b).wait()


def kernel(input_ids, logits, logits_mask):
    masked, markers = _sc_kernel(logits, logits_mask)
    return masked, markers.astype(input_ids.dtype)
